# baseline (speedup 1.0000x reference)
import functools

import jax
import jax.numpy as jnp
from jax import lax
from jax.experimental import pallas as pl
from jax.experimental.pallas import tpu as pltpu
from jax.experimental.pallas import tpu_sc as plsc

NC, NS = 2, 16
NW = NC * NS
LANES = 16
CHUNK = 128
NBUF = 5
GAT_AHEAD = 3


@functools.lru_cache(maxsize=None)
def _build(batch, seq_len, dim):
    rows = batch * seq_len
    span = seq_len // NW
    nch = (batch * span) // CHUNK
    ch_per_b = span // CHUNK
    nvec = dim // LANES
    mesh = plsc.VectorSubcoreMesh(
        core_axis_name="c", subcore_axis_name="s",
        num_cores=NC, num_subcores=NS)

    @functools.partial(
        pl.kernel,
        out_type=jax.ShapeDtypeStruct((rows, dim), jnp.float32),
        mesh=mesh,
        scratch_types=[
            pltpu.VMEM((batch, span), jnp.int32),
            pltpu.VMEM((span, dim), jnp.float32),
            pltpu.VMEM((NBUF, CHUNK, dim), jnp.float32),
            pltpu.SemaphoreType.DMA,
            pltpu.SemaphoreType.DMA,
            pltpu.SemaphoreType.DMA((NBUF,)),
            pltpu.SemaphoreType.DMA((NBUF,)),
        ],
    )
    def emb(ids_hbm, table_hbm, pos_hbm, out_hbm, idx_v, pos_v, bufs,
            id_sem, ld_sem, gat_sem, out_sem):
        wid = lax.axis_index("s") * NC + lax.axis_index("c")
        s0 = wid * span

        pos_ld = pltpu.async_copy(pos_hbm.at[pl.ds(s0, span)], pos_v, ld_sem)
        with jax.named_scope("idxwait"):
            pltpu.async_copy(ids_hbm.at[:, pl.ds(s0, span)], idx_v,
                             id_sem).wait()

        pend_gat, pend_out = {}, {}

        def flat_base(c):
            b_row, half = divmod(c, ch_per_b)
            return b_row * seq_len + s0 + half * CHUNK

        def start_gather(c):
            b = c % NBUF
            if c - NBUF in pend_out:
                pend_out.pop(c - NBUF).wait()
            b_row, half = divmod(c, ch_per_b)
            pend_gat[c] = pltpu.async_copy(
                table_hbm.at[idx_v.at[b_row, pl.ds(half * CHUNK, CHUNK)]],
                bufs.at[b], gat_sem.at[b])

        def add_pos(c):
            b = c % NBUF
            half = c % ch_per_b
            buf = bufs.at[b]

            def body(r, carry):
                pr = half * CHUNK + r
                for j in range(nvec):
                    sl = pl.ds(j * LANES, LANES)
                    plsc.addupdate(buf.at[r, sl], pos_v[pr, sl])
                return carry

            lax.fori_loop(0, CHUNK, body, 0)

        def finish_chunk(c):
            b = c % NBUF
            with jax.named_scope(f"gwait{c}"):
                pend_gat.pop(c).wait()
            with jax.named_scope(f"add{c}"):
                add_pos(c)
            pend_out[c] = pltpu.async_copy(
                bufs.at[b], out_hbm.at[pl.ds(flat_base(c), CHUNK)],
                out_sem.at[b])

        for c in range(GAT_AHEAD):
            start_gather(c)
        for c in range(nch):
            if c + GAT_AHEAD < nch:
                start_gather(c + GAT_AHEAD)
            if c == 0:
                pos_ld.wait()
            finish_chunk(c)
        for c in sorted(pend_out):
            pend_out.pop(c).wait()

    return emb


def kernel(input_ids, embedding_weight, pos_embedding):
    batch, seq_len = input_ids.shape
    _, dim = embedding_weight.shape
    ids = input_ids.astype(jnp.int32)
    out = _build(batch, seq_len, dim)(ids, embedding_weight, pos_embedding)
    return out.reshape(batch, seq_len, dim)


# speedup vs baseline: 1.0087x; 1.0087x over previous
import functools

import jax
import jax.numpy as jnp
from jax import lax
from jax.experimental import pallas as pl
from jax.experimental.pallas import tpu as pltpu
from jax.experimental.pallas import tpu_sc as plsc

NC, NS = 2, 16
NW = NC * NS
LANES = 16
CHUNK = 128
NBUF = 5
GAT_AHEAD = 3


@functools.lru_cache(maxsize=None)
def _build(batch, seq_len, dim):
    rows = batch * seq_len
    span = seq_len // NW
    nch = (batch * span) // CHUNK
    ch_per_b = span // CHUNK
    nvec = dim // LANES
    mesh = plsc.VectorSubcoreMesh(
        core_axis_name="c", subcore_axis_name="s",
        num_cores=NC, num_subcores=NS)

    @functools.partial(
        pl.kernel,
        out_type=jax.ShapeDtypeStruct((rows, dim), jnp.float32),
        mesh=mesh,
        scratch_types=[
            pltpu.VMEM((batch, span), jnp.int32),
            pltpu.VMEM((span, dim), jnp.float32),
            pltpu.VMEM((NBUF, CHUNK, dim), jnp.float32),
            pltpu.SemaphoreType.DMA,
            pltpu.SemaphoreType.DMA((2,)),
            pltpu.SemaphoreType.DMA((NBUF,)),
            pltpu.SemaphoreType.DMA((NBUF,)),
        ],
    )
    def emb(ids_hbm, table_hbm, pos_hbm, out_hbm, idx_v, pos_v, bufs,
            id_sem, ld_sem, gat_sem, out_sem):
        wid = lax.axis_index("s") * NC + lax.axis_index("c")
        s0 = wid * span

        pos_lds = [
            pltpu.async_copy(pos_hbm.at[pl.ds(s0 + h * CHUNK, CHUNK)],
                             pos_v.at[pl.ds(h * CHUNK, CHUNK)], ld_sem.at[h])
            for h in range(ch_per_b)]
        with jax.named_scope("idxwait"):
            pltpu.async_copy(ids_hbm.at[:, pl.ds(s0, span)], idx_v,
                             id_sem).wait()

        pend_gat, pend_out = {}, {}

        def chunk_coords(c):
            half, b_row = divmod(c, batch)
            return b_row, half

        def flat_base(c):
            b_row, half = chunk_coords(c)
            return b_row * seq_len + s0 + half * CHUNK

        def start_gather(c):
            b = c % NBUF
            if c - NBUF in pend_out:
                pend_out.pop(c - NBUF).wait()
            b_row, half = chunk_coords(c)
            pend_gat[c] = pltpu.async_copy(
                table_hbm.at[idx_v.at[b_row, pl.ds(half * CHUNK, CHUNK)]],
                bufs.at[b], gat_sem.at[b])

        def add_pos(c):
            b = c % NBUF
            half = chunk_coords(c)[1]
            buf = bufs.at[b]

            def body(r, carry):
                pr = half * CHUNK + r
                for j in range(nvec):
                    sl = pl.ds(j * LANES, LANES)
                    plsc.addupdate(buf.at[r, sl], pos_v[pr, sl])
                return carry

            lax.fori_loop(0, CHUNK, body, 0)

        def finish_chunk(c):
            b = c % NBUF
            half = chunk_coords(c)[1]
            if pos_lds[half] is not None:
                with jax.named_scope(f"poswait{half}"):
                    pos_lds[half].wait()
                pos_lds[half] = None
            with jax.named_scope(f"gwait{c}"):
                pend_gat.pop(c).wait()
            with jax.named_scope(f"add{c}"):
                add_pos(c)
            pend_out[c] = pltpu.async_copy(
                bufs.at[b], out_hbm.at[pl.ds(flat_base(c), CHUNK)],
                out_sem.at[b])

        for c in range(GAT_AHEAD):
            start_gather(c)
        for c in range(nch):
            if c + GAT_AHEAD < nch:
                start_gather(c + GAT_AHEAD)
            finish_chunk(c)
        for c in sorted(pend_out):
            pend_out.pop(c).wait()

    return emb


def kernel(input_ids, embedding_weight, pos_embedding):
    batch, seq_len = input_ids.shape
    _, dim = embedding_weight.shape
    ids = input_ids.astype(jnp.int32)
    out = _build(batch, seq_len, dim)(ids, embedding_weight, pos_embedding)
    return out.reshape(batch, seq_len, dim)


# speedup vs baseline: 1.0306x; 1.0216x over previous
import functools

import jax
import jax.numpy as jnp
from jax import lax
from jax.experimental import pallas as pl
from jax.experimental.pallas import tpu as pltpu
from jax.experimental.pallas import tpu_sc as plsc

NC, NS = 2, 16
NW = NC * NS
LANES = 16
CHUNK = 128
NBUF = 5
GAT_AHEAD = 3


@functools.lru_cache(maxsize=None)
def _build(batch, seq_len, dim):
    rows = batch * seq_len
    span = seq_len // NW
    nch = (batch * span) // CHUNK
    ch_per_b = span // CHUNK
    nvec = dim // LANES
    mesh = plsc.VectorSubcoreMesh(
        core_axis_name="c", subcore_axis_name="s",
        num_cores=NC, num_subcores=NS)

    @functools.partial(
        pl.kernel,
        out_type=jax.ShapeDtypeStruct((rows, dim), jnp.float32),
        mesh=mesh,
        scratch_types=[
            pltpu.VMEM((batch, span), jnp.int32),
            pltpu.VMEM((span, dim), jnp.float32),
            pltpu.VMEM((NBUF, CHUNK, dim), jnp.float32),
            pltpu.SemaphoreType.DMA,
            pltpu.SemaphoreType.DMA((2,)),
            pltpu.SemaphoreType.DMA((NBUF,)),
            pltpu.SemaphoreType.DMA((NBUF,)),
        ],
    )
    def emb(ids_hbm, table_hbm, pos_hbm, out_hbm, idx_v, pos_v, bufs,
            id_sem, ld_sem, gat_sem, out_sem):
        wid = lax.axis_index("s") * NC + lax.axis_index("c")
        s0 = wid * span

        pos_lds = [
            pltpu.async_copy(pos_hbm.at[pl.ds(s0 + h * CHUNK, CHUNK)],
                             pos_v.at[pl.ds(h * CHUNK, CHUNK)], ld_sem.at[h])
            for h in range(ch_per_b)]
        with jax.named_scope("idxwait"):
            pltpu.async_copy(ids_hbm.at[:, pl.ds(s0, span)], idx_v,
                             id_sem).wait()

        pend_gat, pend_out = {}, {}

        chunks = [(0, 0, 32), (0, 32, 96)]
        chunks += [(b_row, 0, CHUNK) for b_row in range(1, batch)]
        chunks += [(b_row, CHUNK, CHUNK) for b_row in range(batch)]
        ncs = len(chunks)

        def start_gather(c):
            b = c % NBUF
            if c - NBUF in pend_out:
                pend_out.pop(c - NBUF).wait()
            b_row, roff, n = chunks[c]
            pend_gat[c] = pltpu.async_copy(
                table_hbm.at[idx_v.at[b_row, pl.ds(roff, n)]],
                bufs.at[b, pl.ds(0, n)], gat_sem.at[b])

        def add_pos(c):
            b = c % NBUF
            _, roff, n = chunks[c]
            buf = bufs.at[b]

            def body(r, carry):
                pr = roff + r
                for j in range(nvec):
                    sl = pl.ds(j * LANES, LANES)
                    plsc.addupdate(buf.at[r, sl], pos_v[pr, sl])
                return carry

            lax.fori_loop(0, n, body, 0)

        def finish_chunk(c):
            b = c % NBUF
            b_row, roff, n = chunks[c]
            half = roff // CHUNK
            if pos_lds[half] is not None:
                with jax.named_scope(f"poswait{half}"):
                    pos_lds[half].wait()
                pos_lds[half] = None
            with jax.named_scope(f"gwait{c}"):
                pend_gat.pop(c).wait()
            with jax.named_scope(f"add{c}"):
                add_pos(c)
            pend_out[c] = pltpu.async_copy(
                bufs.at[b, pl.ds(0, n)],
                out_hbm.at[pl.ds(b_row * seq_len + s0 + roff, n)],
                out_sem.at[b])

        for c in range(GAT_AHEAD):
            start_gather(c)
        for c in range(ncs):
            if c + GAT_AHEAD < ncs:
                start_gather(c + GAT_AHEAD)
            finish_chunk(c)
        for c in sorted(pend_out):
            pend_out.pop(c).wait()

    return emb


def kernel(input_ids, embedding_weight, pos_embedding):
    batch, seq_len = input_ids.shape
    _, dim = embedding_weight.shape
    ids = input_ids.astype(jnp.int32)
    out = _build(batch, seq_len, dim)(ids, embedding_weight, pos_embedding)
    return out.reshape(batch, seq_len, dim)
